# baseline (speedup 1.0000x reference)
import functools
import math

import jax
import jax.numpy as jnp
from jax import lax
from jax.experimental import pallas as pl
from jax.experimental.pallas import tpu as pltpu
from jax.experimental.pallas import tpu_sc as plsc

N = 10000
E = 320000
D = 480
DP = 512
CH = 4
CW = 128
DF = 1920
NBF = 32
L = 6
G = 512
MAX_R = 5.0
AVG_DEGREE = 15.57930850982666
AVG_NUM_NODES = 18.03065905448718

NCORES = 2
NSUB = 16
NW = NCORES * NSUB
EPW = E // NW
EPT = (E // 2) // NSUB
KE = 80
KG = 80
FG = 80
NPAD = 10240
NPT = NPAD // NW
NACC = NPAD
RPT = NACC // NSUB
FB = 64
NFL = RPT // FB

_INV_SQRT_DEG = 1.0 / math.sqrt(AVG_DEGREE)
_INV_SQRT_NODES = 1.0 / math.sqrt(AVG_NUM_NODES)
_S3 = math.sqrt(3.0)
_S5 = math.sqrt(5.0)
_S15 = math.sqrt(15.0)


def _sc_mesh():
    return plsc.VectorSubcoreMesh(core_axis_name="c", subcore_axis_name="s",
                                  num_cores=NCORES, num_subcores=NSUB)


def _sc_edges_atoms_body(posx, posy, posz, src, dst, zp, atab,
                         evx, evy, evz, xemb,
                         px, py, pz, si, di, obx, oby, obz, zi, xb, sem):
    cid = lax.axis_index("c")
    sid = lax.axis_index("s")
    wid = sid * NCORES + cid
    ebase = wid * EPW
    pltpu.sync_copy(posx, px)
    pltpu.sync_copy(posy, py)
    pltpu.sync_copy(posz, pz)

    def eblk(b, _):
        e0 = ebase + b * KE
        pltpu.sync_copy(src.at[pl.ds(e0, KE)], si)
        pltpu.sync_copy(dst.at[pl.ds(e0, KE)], di)
        for g in range(KE // 16):
            s = pl.ds(g * 16, 16)
            sg = si[s]
            dg = di[s]
            obx[s] = plsc.load_gather(px, [sg]) - plsc.load_gather(px, [dg])
            oby[s] = plsc.load_gather(py, [sg]) - plsc.load_gather(py, [dg])
            obz[s] = plsc.load_gather(pz, [sg]) - plsc.load_gather(pz, [dg])
        pltpu.sync_copy(obx, evx.at[pl.ds(e0, KE)])
        pltpu.sync_copy(oby, evy.at[pl.ds(e0, KE)])
        pltpu.sync_copy(obz, evz.at[pl.ds(e0, KE)])
        return 0

    lax.fori_loop(0, EPW // KE, eblk, 0)

    nbase = wid * NPT

    def nblk(b, _):
        n0 = nbase + b * KE
        pltpu.sync_copy(zp.at[pl.ds(n0, KE)], zi)
        pltpu.async_copy(atab.at[zi], xb, sem).wait()
        pltpu.sync_copy(xb, xemb.at[pl.ds(n0, KE)])
        return 0

    lax.fori_loop(0, NPT // KE, nblk, 0)


def _sc_edges_atoms(posx, posy, posz, src, dst, zp, atabp):
    kfn = pl.kernel(
        _sc_edges_atoms_body,
        out_type=(
            jax.ShapeDtypeStruct((E,), jnp.float32),
            jax.ShapeDtypeStruct((E,), jnp.float32),
            jax.ShapeDtypeStruct((E,), jnp.float32),
            jax.ShapeDtypeStruct((NPAD, DP), jnp.float32),
        ),
        mesh=_sc_mesh(),
        scratch_types=[
            pltpu.VMEM((N,), jnp.float32),
            pltpu.VMEM((N,), jnp.float32),
            pltpu.VMEM((N,), jnp.float32),
            pltpu.VMEM((KE,), jnp.int32),
            pltpu.VMEM((KE,), jnp.int32),
            pltpu.VMEM((KE,), jnp.float32),
            pltpu.VMEM((KE,), jnp.float32),
            pltpu.VMEM((KE,), jnp.float32),
            pltpu.VMEM((KE,), jnp.int32),
            pltpu.VMEM((KE, DP), jnp.float32),
            pltpu.SemaphoreType.DMA,
        ],
        compiler_params=pltpu.CompilerParams(needs_layout_passes=False),
    )
    return kfn(posx, posy, posz, src, dst, zp, atabp)


def _zero_buf(zb):
    def zrow(r, _):
        for g in range(CW // 16):
            zb[r, pl.ds(g * 16, 16)] = jnp.zeros((16,), jnp.float32)
        return 0

    lax.fori_loop(0, FB, zrow, 0)


def _sc_scatter_body(f0, f1, f2, f3, dst, out,
                     di0, di1, fb0, fb1, zfb, acc, sem0, sem1):
    cid = lax.axis_index("c")
    sid = lax.axis_index("s")
    ebase = cid * (E // 2) + sid * EPT
    feats = [f0, f1, f2, f3]
    dis = [di0, di1]
    fbs = [fb0, fb1]
    sems = [sem0, sem1]
    nb = EPT // KE
    for k in range(CH):
        _zero_buf(zfb)

        def zflush(j, _):
            r0 = pl.multiple_of(sid * RPT + j * FB, 8)
            pltpu.sync_copy(zfb, acc.at[pl.ds(r0, FB)])
            return 0

        lax.fori_loop(0, NFL, zflush, 0)
        plsc.subcore_barrier()
        fk = feats[k]

        def issue(b, p):
            e0 = ebase + b * KE
            pltpu.async_copy(dst.at[pl.ds(e0, KE)], dis[p], sems[p])
            pltpu.async_copy(fk.at[pl.ds(e0, KE)], fbs[p], sems[p])

        def drain(b, p):
            e0 = ebase + b * KE
            pltpu.make_async_copy(dst.at[pl.ds(e0, KE)], dis[p], sems[p]).wait()
            pltpu.make_async_copy(fk.at[pl.ds(e0, KE)], fbs[p], sems[p]).wait()

        def step(b, p, pre):
            drain(b, p)
            pltpu.sync_copy(fbs[p], acc.at[dis[p]], add=True)
            if pre:
                issue(b + 2, p)

        issue(0, 0)
        issue(1, 1)
        step(0, 0, True)

        def pair(t, _):
            b = 2 * t + 1
            step(b, 1, True)
            step(b + 1, 0, True)
            return 0

        lax.fori_loop(0, (nb - 3) // 2, pair, 0)
        step(nb - 2, 1, False)
        step(nb - 1, 0, False)
        plsc.subcore_barrier()

        def fflush(j, _):
            r0 = pl.multiple_of(sid * RPT + j * FB, 8)
            pltpu.sync_copy(acc.at[pl.ds(r0, FB)], zfb)
            off = pl.multiple_of((cid * CH + k) * NACC + r0, 8)
            pltpu.sync_copy(zfb, out.at[pl.ds(off, FB)])
            return 0

        lax.fori_loop(0, NFL, fflush, 0)


def _sc_scatter(f_chunks, dst):
    kfn = pl.kernel(
        _sc_scatter_body,
        out_type=jax.ShapeDtypeStruct((2 * CH * NACC, CW), jnp.float32),
        mesh=_sc_mesh(),
        scratch_types=[
            pltpu.VMEM((KE,), jnp.int32),
            pltpu.VMEM((KE,), jnp.int32),
            pltpu.VMEM((KE, CW), jnp.float32),
            pltpu.VMEM((KE, CW), jnp.float32),
            pltpu.VMEM((FB, CW), jnp.float32),
            pltpu.VMEM_SHARED((NACC, CW), jnp.float32),
            pltpu.SemaphoreType.DMA,
            pltpu.SemaphoreType.DMA,
        ],
    )
    return kfn(f_chunks[0], f_chunks[1], f_chunks[2], f_chunks[3], dst)


def _sc_gather_mul_scatter_body(g0, g1, g2, g3, y0, y1, y2, y3, src, dst, out,
                                si0, si1, di0, di1, gb0, gb1, yb0, yb1, acc,
                                semi0, semi1, semy0, semy1, sems0, sems1):
    cid = lax.axis_index("c")
    sid = lax.axis_index("s")
    ebase = cid * (E // 2) + sid * EPT
    gates = [g0, g1, g2, g3]
    ys = [y0, y1, y2, y3]
    sis = [si0, si1]
    dis = [di0, di1]
    gbs = [gb0, gb1]
    ybs = [yb0, yb1]
    semis = [semi0, semi1]
    semys = [semy0, semy1]
    semss = [sems0, sems1]
    nb = EPT // KG
    nfl = RPT // FG
    for k in range(CH):
        def zrow(r, _):
            for g in range(CW // 16):
                yb0[r, pl.ds(g * 16, 16)] = jnp.zeros((16,), jnp.float32)
            return 0

        lax.fori_loop(0, FG, zrow, 0)

        def zflush(j, _):
            r0 = pl.multiple_of(sid * RPT + j * FG, 8)
            pltpu.sync_copy(yb0, acc.at[pl.ds(r0, FG)])
            return 0

        lax.fori_loop(0, nfl, zflush, 0)
        plsc.subcore_barrier()
        gk = gates[k]
        yk = ys[k]

        def issue_idx(b, p):
            e0 = ebase + b * KG
            g0p = pl.multiple_of(ebase // 2 + b * (KG // 2), 8)
            pltpu.async_copy(src.at[pl.ds(e0, KG)], sis[p], semis[p])
            pltpu.async_copy(dst.at[pl.ds(e0, KG)], dis[p], semis[p])
            pltpu.async_copy(gk.at[pl.ds(g0p, KG // 2)], gbs[p], semis[p])

        def drain_idx(b, p):
            e0 = ebase + b * KG
            g0p = pl.multiple_of(ebase // 2 + b * (KG // 2), 8)
            pltpu.make_async_copy(src.at[pl.ds(e0, KG)], sis[p], semis[p]).wait()
            pltpu.make_async_copy(dst.at[pl.ds(e0, KG)], dis[p], semis[p]).wait()
            pltpu.make_async_copy(gk.at[pl.ds(g0p, KG // 2)], gbs[p],
                                  semis[p]).wait()

        def issue_y(p):
            pltpu.async_copy(yk.at[sis[p]], ybs[p], semys[p])

        def drain_y(p):
            pltpu.make_async_copy(yk.at[sis[p]], ybs[p], semys[p]).wait()

        def drain_scatter(p):
            pltpu.make_async_copy(ybs[p], acc.at[dis[p]], semss[p]).wait()

        def mul_scatter(p):
            yb = ybs[p]
            gb = gbs[p]

            def row(r, _):
                for rr in range(2):
                    for g in range(CW // 16):
                        s = pl.ds(g * 16, 16)
                        gv = gb[r, rr, s].astype(jnp.float32)
                        yb[2 * r + rr, s] = yb[2 * r + rr, s] * gv
                return 0

            lax.fori_loop(0, KG // 2, row, 0)
            pltpu.async_copy(yb, acc.at[dis[p]], semss[p], add=True)

        def step(b, p, pre_idx, pre_y, drain_sc):
            drain_y(p)
            if drain_sc:
                drain_scatter(1 - p)
            if pre_y:
                drain_idx(b + 1, 1 - p)
                issue_y(1 - p)
            mul_scatter(p)
            if pre_idx:
                issue_idx(b + 2, p)

        issue_idx(0, 0)
        drain_idx(0, 0)
        issue_y(0)
        issue_idx(1, 1)
        step(0, 0, True, True, False)

        def pair(t, _):
            b = 2 * t + 1
            step(b, 1, True, True, True)
            step(b + 1, 0, True, True, True)
            return 0

        lax.fori_loop(0, (nb - 3) // 2, pair, 0)
        step(nb - 2, 1, False, True, True)
        step(nb - 1, 0, False, False, True)
        drain_scatter(0)
        plsc.subcore_barrier()

        def fflush(j, _):
            r0 = pl.multiple_of(sid * RPT + j * FG, 8)
            pltpu.sync_copy(acc.at[pl.ds(r0, FG)], yb0)
            off = pl.multiple_of((cid * CH + k) * NACC + r0, 8)
            pltpu.sync_copy(yb0, out.at[pl.ds(off, FG)])
            return 0

        lax.fori_loop(0, nfl, fflush, 0)


def _sc_gather_mul_scatter(g_chunks, y_chunks, src, dst):
    kfn = pl.kernel(
        _sc_gather_mul_scatter_body,
        out_type=jax.ShapeDtypeStruct((2 * CH * NACC, CW), jnp.float32),
        mesh=_sc_mesh(),
        scratch_types=[
            pltpu.VMEM((KG,), jnp.int32),
            pltpu.VMEM((KG,), jnp.int32),
            pltpu.VMEM((KG,), jnp.int32),
            pltpu.VMEM((KG,), jnp.int32),
            pltpu.VMEM((KG // 2, 2, CW), jnp.bfloat16),
            pltpu.VMEM((KG // 2, 2, CW), jnp.bfloat16),
            pltpu.VMEM((KG, CW), jnp.float32),
            pltpu.VMEM((KG, CW), jnp.float32),
            pltpu.VMEM_SHARED((NACC, CW), jnp.float32),
            pltpu.SemaphoreType.DMA,
            pltpu.SemaphoreType.DMA,
            pltpu.SemaphoreType.DMA,
            pltpu.SemaphoreType.DMA,
            pltpu.SemaphoreType.DMA,
            pltpu.SemaphoreType.DMA,
        ],
    )
    return kfn(g_chunks[0], g_chunks[1], g_chunks[2], g_chunks[3],
               y_chunks[0], y_chunks[1], y_chunks[2], y_chunks[3], src, dst)


def _silu(v):
    return v * jax.nn.sigmoid(v)


def _sh16(vec16):
    b = vec16.shape[0]
    r = jnp.sqrt(jnp.sum(vec16 * vec16, axis=1, keepdims=True))
    u = vec16 / jnp.maximum(r, 1e-9)
    x = u[:, 0:1]
    y = u[:, 1:2]
    z = u[:, 2:3]
    sh = jnp.concatenate([
        jnp.ones((b, 1), jnp.float32),
        _S3 * x, _S3 * y, _S3 * z,
        _S15 * x * y, _S15 * y * z,
        (_S5 / 2.0) * (3.0 * z * z - 1.0),
        _S15 * x * z, (_S15 / 2.0) * (x * x - y * y),
        jnp.zeros((b, 7), jnp.float32),
    ], axis=1)
    return sh, r


def _tc_edge_feat_body(ev_ref, wd1_ref, wd2_ref, wsh_ref,
                       f0_ref, f1_ref, f2_ref, f3_ref, rbf_ref):
    ev = ev_ref[...]
    sh, d = _sh16(ev)
    e5 = math.exp(-MAX_R)
    means = e5 + lax.broadcasted_iota(jnp.int32, (1, NBF), 1).astype(
        jnp.float32) * ((1.0 - e5) / (NBF - 1))
    beta = (2.0 / NBF * (1.0 - e5)) ** -2
    fc = 0.5 * (jnp.cos(jnp.pi * jnp.clip(d, 0.0, MAX_R) / MAX_R) + 1.0)
    rbf = fc * jnp.exp(-beta * (jnp.exp(-d) - means) ** 2)
    rbf_ref[...] = rbf
    h = _silu(jnp.dot(rbf, wd1_ref[...], preferred_element_type=jnp.float32))
    dw = jnp.dot(h, wd2_ref[...], preferred_element_type=jnp.float32)
    f = dw * jnp.dot(sh, wsh_ref[...], preferred_element_type=jnp.float32)
    f0_ref[...] = f[:, 0 * CW:1 * CW]
    f1_ref[...] = f[:, 1 * CW:2 * CW]
    f2_ref[...] = f[:, 2 * CW:3 * CW]
    f3_ref[...] = f[:, 3 * CW:4 * CW]


def _tc_edge_feat(evec, wd1, wd2p, wshp):
    be = 512
    fspec = pl.BlockSpec((be, CW), lambda i: (i, 0))
    return pl.pallas_call(
        _tc_edge_feat_body,
        grid=(E // be,),
        in_specs=[
            pl.BlockSpec((be, 16), lambda i: (i, 0)),
            pl.BlockSpec((NBF, 64), lambda i: (0, 0)),
            pl.BlockSpec((64, DP), lambda i: (0, 0)),
            pl.BlockSpec((16, DP), lambda i: (0, 0)),
        ],
        out_specs=[fspec, fspec, fspec, fspec,
                   pl.BlockSpec((be, NBF), lambda i: (i, 0))],
        out_shape=[jax.ShapeDtypeStruct((E, CW), jnp.float32) for _ in range(4)]
        + [jax.ShapeDtypeStruct((E, NBF), jnp.float32)],
    )(evec, wd1, wd2p, wshp)


def _tc_gate_body(rbf_ref, wr1_ref, wr2_ref, g0_ref, g1_ref, g2_ref, g3_ref):
    h = _silu(jnp.dot(rbf_ref[...], wr1_ref[...],
                      preferred_element_type=jnp.float32))
    gfull = _silu(jnp.dot(h, wr2_ref[...], preferred_element_type=jnp.float32))
    gfull = gfull.astype(jnp.bfloat16)
    be = gfull.shape[0]
    g0_ref[...] = gfull[:, 0 * CW:1 * CW].reshape(be // 2, 2, CW)
    g1_ref[...] = gfull[:, 1 * CW:2 * CW].reshape(be // 2, 2, CW)
    g2_ref[...] = gfull[:, 2 * CW:3 * CW].reshape(be // 2, 2, CW)
    g3_ref[...] = gfull[:, 3 * CW:4 * CW].reshape(be // 2, 2, CW)


def _tc_gate(rbf, wr1_i, wr2p_i):
    be = 512
    gspec = pl.BlockSpec((be // 2, 2, CW), lambda i: (i, 0, 0))
    return pl.pallas_call(
        _tc_gate_body,
        grid=(E // be,),
        in_specs=[
            pl.BlockSpec((be, NBF), lambda i: (i, 0)),
            pl.BlockSpec((NBF, 64), lambda i: (0, 0)),
            pl.BlockSpec((64, DP), lambda i: (0, 0)),
        ],
        out_specs=[gspec] * 4,
        out_shape=[jax.ShapeDtypeStruct((E // 2, 2, CW), jnp.bfloat16)
                   for _ in range(4)],
    )(rbf, wr1_i, wr2p_i)


def _tc_x0_body(xemb_ref, part_ref, force_ref, wf_ref, wm_ref,
                x_ref, y0_ref, y1_ref, y2_ref, y3_ref):
    p = part_ref[...]
    agg = jnp.concatenate([p[0, k] + p[1, k] for k in range(CH)], axis=1)
    sh, r = _sh16(force_ref[...])
    fenc = jnp.dot(sh * (r * (1.0 / _S3)), wf_ref[...],
                   preferred_element_type=jnp.float32)
    x = 8.0 * xemb_ref[...] + agg * _INV_SQRT_DEG + fenc
    x_ref[...] = x
    y = jnp.dot(x, wm_ref[...], preferred_element_type=jnp.float32)
    y0_ref[...] = y[:, 0 * CW:1 * CW]
    y1_ref[...] = y[:, 1 * CW:2 * CW]
    y2_ref[...] = y[:, 2 * CW:3 * CW]
    y3_ref[...] = y[:, 3 * CW:4 * CW]


def _tc_x0(xemb, part, forcep, wfp, wm0):
    bn = 400
    yspec = pl.BlockSpec((bn, CW), lambda i: (i, 0))
    return pl.pallas_call(
        _tc_x0_body,
        grid=(N // bn,),
        in_specs=[
            pl.BlockSpec((bn, DP), lambda i: (i, 0)),
            pl.BlockSpec((2, CH, bn, CW), lambda i: (0, 0, i, 0)),
            pl.BlockSpec((bn, 16), lambda i: (i, 0)),
            pl.BlockSpec((16, DP), lambda i: (0, 0)),
            pl.BlockSpec((DP, DP), lambda i: (0, 0)),
        ],
        out_specs=[pl.BlockSpec((bn, DP), lambda i: (i, 0))] + [yspec] * 4,
        out_shape=[jax.ShapeDtypeStruct((N, DP), jnp.float32)]
        + [jax.ShapeDtypeStruct((N, CW), jnp.float32) for _ in range(4)],
    )(xemb, part, forcep, wfp, wm0)


def _tc_update_body(x_ref, part_ref, wu_ref, wm_ref,
                    xn_ref, y0_ref, y1_ref, y2_ref, y3_ref):
    p = part_ref[...]
    agg = jnp.concatenate([p[0, k] + p[1, k] for k in range(CH)], axis=1)
    a = _silu(agg * _INV_SQRT_DEG)
    xn = x_ref[...] + jnp.dot(a, wu_ref[...], preferred_element_type=jnp.float32)
    xn_ref[...] = xn
    y = jnp.dot(xn, wm_ref[...], preferred_element_type=jnp.float32)
    y0_ref[...] = y[:, 0 * CW:1 * CW]
    y1_ref[...] = y[:, 1 * CW:2 * CW]
    y2_ref[...] = y[:, 2 * CW:3 * CW]
    y3_ref[...] = y[:, 3 * CW:4 * CW]


def _tc_update(x, part, wu_i, wm_next):
    bn = 400
    yspec = pl.BlockSpec((bn, CW), lambda i: (i, 0))
    return pl.pallas_call(
        _tc_update_body,
        grid=(N // bn,),
        in_specs=[
            pl.BlockSpec((bn, DP), lambda i: (i, 0)),
            pl.BlockSpec((2, CH, bn, CW), lambda i: (0, 0, i, 0)),
            pl.BlockSpec((DP, DP), lambda i: (0, 0)),
            pl.BlockSpec((DP, DP), lambda i: (0, 0)),
        ],
        out_specs=[pl.BlockSpec((bn, DP), lambda i: (i, 0))] + [yspec] * 4,
        out_shape=[jax.ShapeDtypeStruct((N, DP), jnp.float32)]
        + [jax.ShapeDtypeStruct((N, CW), jnp.float32) for _ in range(4)],
    )(x, part, wu_i, wm_next)


def _tc_update_last_body(x_ref, part_ref, wu_ref, xn_ref):
    p = part_ref[...]
    agg = jnp.concatenate([p[0, k] + p[1, k] for k in range(CH)], axis=1)
    a = _silu(agg * _INV_SQRT_DEG)
    xn_ref[...] = x_ref[...] + jnp.dot(a, wu_ref[...],
                                       preferred_element_type=jnp.float32)


def _tc_update_last(x, part, wu_i):
    bn = 400
    return pl.pallas_call(
        _tc_update_last_body,
        grid=(N // bn,),
        in_specs=[
            pl.BlockSpec((bn, DP), lambda i: (i, 0)),
            pl.BlockSpec((2, CH, bn, CW), lambda i: (0, 0, i, 0)),
            pl.BlockSpec((DP, DP), lambda i: (0, 0)),
        ],
        out_specs=pl.BlockSpec((bn, DP), lambda i: (i, 0)),
        out_shape=jax.ShapeDtypeStruct((N, DP), jnp.float32),
    )(x, part, wu_i)


def _tc_head_body(x_ref, batch_ref, wfeat_ref, we1_ref, we2_ref, wp_ref,
                  energy_ref, pos_ref):
    feat = jnp.dot(x_ref[...], wfeat_ref[...], preferred_element_type=jnp.float32)
    rms = jnp.sqrt(jnp.mean(feat * feat, axis=1, keepdims=True) + 1e-6)
    feat = feat / rms
    scal = feat[:, :512]
    h = _silu(jnp.dot(scal, we1_ref[...], preferred_element_type=jnp.float32))
    node_e = jnp.dot(h, we2_ref[...], preferred_element_type=jnp.float32)
    node_e = node_e * _INV_SQRT_NODES
    seg = lax.broadcasted_iota(jnp.int32, (G, 1), 0)
    onehot = (seg == batch_ref[0]).astype(jnp.float32)
    epart = jnp.dot(onehot, node_e, preferred_element_type=jnp.float32)

    @pl.when(pl.program_id(0) == 0)
    def _():
        energy_ref[...] = epart

    @pl.when(pl.program_id(0) > 0)
    def _():
        energy_ref[...] += epart

    pos_ref[...] = jnp.dot(feat, wp_ref[...], preferred_element_type=jnp.float32)


def _tc_head(x, batchp, wfeatp, we1, we2p, wpp):
    bn = 400
    return pl.pallas_call(
        _tc_head_body,
        grid=(N // bn,),
        in_specs=[
            pl.BlockSpec((bn, DP), lambda i: (i, 0)),
            pl.BlockSpec((1, 1, bn), lambda i: (i, 0, 0)),
            pl.BlockSpec((DP, DF), lambda i: (0, 0)),
            pl.BlockSpec((512, 512), lambda i: (0, 0)),
            pl.BlockSpec((512, CW), lambda i: (0, 0)),
            pl.BlockSpec((DF, CW), lambda i: (0, 0)),
        ],
        out_specs=[
            pl.BlockSpec((G, CW), lambda i: (0, 0)),
            pl.BlockSpec((bn, CW), lambda i: (i, 0)),
        ],
        out_shape=[
            jax.ShapeDtypeStruct((G, CW), jnp.float32),
            jax.ShapeDtypeStruct((N, CW), jnp.float32),
        ],
    )(x, batchp, wfeatp, we1, we2p, wpp)


def kernel(z, pos, batch, edge_index, force, noise_mask, atom_table, W_force,
           W_sh, Wd1, Wd2, Wr1, Wr2, Wmsg, Wupd, Wfeat, We1, We2, Wp):
    f32 = jnp.float32
    src = edge_index[0].astype(jnp.int32)
    dst = edge_index[1].astype(jnp.int32)
    posf = pos.astype(f32)
    posx, posy, posz = posf[:, 0], posf[:, 1], posf[:, 2]
    forcep = jnp.pad(force.astype(f32) * noise_mask[:, None].astype(f32),
                     ((0, 0), (0, 13)))
    zp = jnp.pad(z.astype(jnp.int32), (0, NPAD - N))
    atabp = jnp.pad(atom_table.astype(f32), ((0, 0), (0, DP - D)))
    wshp = jnp.pad(W_sh.astype(f32), ((0, 7), (0, DP - D)))
    wfp = jnp.pad(W_force.astype(f32), ((0, 7), (0, DP - D)))
    wd2p = jnp.pad(Wd2.astype(f32), ((0, 0), (0, DP - D)))
    wr2p = jnp.pad(Wr2.astype(f32), ((0, 0), (0, 0), (0, DP - D)))
    wmsgp = jnp.pad(Wmsg.astype(f32), ((0, 0), (0, DP - D), (0, DP - D)))
    wupdp = jnp.pad(Wupd.astype(f32), ((0, 0), (0, DP - D), (0, DP - D)))
    wfeatp = jnp.pad(Wfeat.astype(f32), ((0, DP - D), (0, 0)))
    we2p = jnp.pad(We2.astype(f32), ((0, 0), (0, CW - 1)))
    wpp = jnp.pad(Wp.astype(f32), ((0, 0), (0, CW - 3)))
    batchp = batch.astype(jnp.int32).reshape(N // 400, 1, 400)

    evx, evy, evz, xemb = _sc_edges_atoms(posx, posy, posz, src, dst, zp, atabp)
    evec = jnp.pad(
        jnp.concatenate([evx[:, None], evy[:, None], evz[:, None]], axis=1),
        ((0, 0), (0, 13)))

    f0, f1, f2, f3, rbf = _tc_edge_feat(evec, Wd1.astype(f32), wd2p, wshp)

    gate_chunks = [_tc_gate(rbf, Wr1[i].astype(f32), wr2p[i]) for i in range(L)]

    deg_part = _sc_scatter((f0, f1, f2, f3), dst).reshape(2, CH, NACC, CW)

    x, y0, y1, y2, y3 = _tc_x0(xemb, deg_part, forcep, wfp, wmsgp[0])
    ys = (y0, y1, y2, y3)

    for i in range(L):
        g_chunks = gate_chunks[i]
        part = _sc_gather_mul_scatter(g_chunks, ys, src, dst)
        part = part.reshape(2, CH, NACC, CW)
        if i < L - 1:
            x, y0, y1, y2, y3 = _tc_update(x, part, wupdp[i], wmsgp[i + 1])
            ys = (y0, y1, y2, y3)
        else:
            x = _tc_update_last(x, part, wupdp[i])

    energy_pad, pos_pad = _tc_head(x, batchp, wfeatp, We1.astype(f32), we2p, wpp)
    return energy_pad[:, :1], pos_pad[:, :3]


# speedup vs baseline: 2.3951x; 2.3951x over previous
import functools
import math

import jax
import jax.numpy as jnp
from jax import lax
from jax.experimental import pallas as pl
from jax.experimental.pallas import tpu as pltpu
from jax.experimental.pallas import tpu_sc as plsc

N = 10000
E = 320000
D = 480
DP = 512
CH = 4
CW = 128
DF = 1920
NBF = 32
L = 6
G = 512
MAX_R = 5.0
AVG_DEGREE = 15.57930850982666
AVG_NUM_NODES = 18.03065905448718

NCORES = 2
NSUB = 16
NW = NCORES * NSUB
EPW = E // NW
EPT = (E // 2) // NSUB
KE = 80
KG = 80
FG = 80
NPAD = 10240
NPT = NPAD // NW
NACC = NPAD
RPT = NACC // NSUB
FB = 64
NFL = RPT // FB

_INV_SQRT_DEG = 1.0 / math.sqrt(AVG_DEGREE)
_INV_SQRT_NODES = 1.0 / math.sqrt(AVG_NUM_NODES)
_S3 = math.sqrt(3.0)
_S5 = math.sqrt(5.0)
_S15 = math.sqrt(15.0)


def _sc_mesh():
    return plsc.VectorSubcoreMesh(core_axis_name="c", subcore_axis_name="s",
                                  num_cores=NCORES, num_subcores=NSUB)


def _sc_edges_atoms_body(posx, posy, posz, src, dst, zp, atab,
                         evx, evy, evz, xemb,
                         px, py, pz, si, di, obx, oby, obz, zi, xb, sem):
    cid = lax.axis_index("c")
    sid = lax.axis_index("s")
    wid = sid * NCORES + cid
    ebase = wid * EPW
    pltpu.sync_copy(posx, px)
    pltpu.sync_copy(posy, py)
    pltpu.sync_copy(posz, pz)

    def eblk(b, _):
        e0 = ebase + b * KE
        pltpu.sync_copy(src.at[pl.ds(e0, KE)], si)
        pltpu.sync_copy(dst.at[pl.ds(e0, KE)], di)
        for g in range(KE // 16):
            s = pl.ds(g * 16, 16)
            sg = si[s]
            dg = di[s]
            obx[s] = plsc.load_gather(px, [sg]) - plsc.load_gather(px, [dg])
            oby[s] = plsc.load_gather(py, [sg]) - plsc.load_gather(py, [dg])
            obz[s] = plsc.load_gather(pz, [sg]) - plsc.load_gather(pz, [dg])
        pltpu.sync_copy(obx, evx.at[pl.ds(e0, KE)])
        pltpu.sync_copy(oby, evy.at[pl.ds(e0, KE)])
        pltpu.sync_copy(obz, evz.at[pl.ds(e0, KE)])
        return 0

    lax.fori_loop(0, EPW // KE, eblk, 0)

    nbase = wid * NPT

    def nblk(b, _):
        n0 = nbase + b * KE
        pltpu.sync_copy(zp.at[pl.ds(n0, KE)], zi)
        pltpu.async_copy(atab.at[zi], xb, sem).wait()
        pltpu.sync_copy(xb, xemb.at[pl.ds(n0, KE)])
        return 0

    lax.fori_loop(0, NPT // KE, nblk, 0)


def _sc_edges_atoms(posx, posy, posz, src, dst, zp, atabp):
    kfn = pl.kernel(
        _sc_edges_atoms_body,
        out_type=(
            jax.ShapeDtypeStruct((E,), jnp.float32),
            jax.ShapeDtypeStruct((E,), jnp.float32),
            jax.ShapeDtypeStruct((E,), jnp.float32),
            jax.ShapeDtypeStruct((NPAD, DP), jnp.float32),
        ),
        mesh=_sc_mesh(),
        scratch_types=[
            pltpu.VMEM((N,), jnp.float32),
            pltpu.VMEM((N,), jnp.float32),
            pltpu.VMEM((N,), jnp.float32),
            pltpu.VMEM((KE,), jnp.int32),
            pltpu.VMEM((KE,), jnp.int32),
            pltpu.VMEM((KE,), jnp.float32),
            pltpu.VMEM((KE,), jnp.float32),
            pltpu.VMEM((KE,), jnp.float32),
            pltpu.VMEM((KE,), jnp.int32),
            pltpu.VMEM((KE, DP), jnp.float32),
            pltpu.SemaphoreType.DMA,
        ],
        compiler_params=pltpu.CompilerParams(needs_layout_passes=False),
    )
    return kfn(posx, posy, posz, src, dst, zp, atabp)


def _zero_buf(zb):
    def zrow(r, _):
        for g in range(CW // 16):
            zb[r, pl.ds(g * 16, 16)] = jnp.zeros((16,), jnp.float32)
        return 0

    lax.fori_loop(0, FB, zrow, 0)


def _sc_scatter_body(f0, f1, f2, f3, dst, out,
                     di0, di1, fb0, fb1, zfb, acc, sem0, sem1):
    cid = lax.axis_index("c")
    sid = lax.axis_index("s")
    ebase = cid * (E // 2) + sid * EPT
    feats = [f0, f1, f2, f3]
    dis = [di0, di1]
    fbs = [fb0, fb1]
    sems = [sem0, sem1]
    nb = EPT // KE
    for k in range(CH):
        _zero_buf(zfb)

        def zflush(j, _):
            r0 = pl.multiple_of(sid * RPT + j * FB, 8)
            pltpu.sync_copy(zfb, acc.at[pl.ds(r0, FB)])
            return 0

        lax.fori_loop(0, NFL, zflush, 0)
        plsc.subcore_barrier()
        fk = feats[k]

        def issue(b, p):
            e0 = ebase + b * KE
            pltpu.async_copy(dst.at[pl.ds(e0, KE)], dis[p], sems[p])
            pltpu.async_copy(fk.at[pl.ds(e0, KE)], fbs[p], sems[p])

        def drain(b, p):
            e0 = ebase + b * KE
            pltpu.make_async_copy(dst.at[pl.ds(e0, KE)], dis[p], sems[p]).wait()
            pltpu.make_async_copy(fk.at[pl.ds(e0, KE)], fbs[p], sems[p]).wait()

        def step(b, p, pre):
            drain(b, p)
            pltpu.sync_copy(fbs[p], acc.at[dis[p]], add=True)
            if pre:
                issue(b + 2, p)

        issue(0, 0)
        issue(1, 1)
        step(0, 0, True)

        def pair(t, _):
            b = 2 * t + 1
            step(b, 1, True)
            step(b + 1, 0, True)
            return 0

        lax.fori_loop(0, (nb - 3) // 2, pair, 0)
        step(nb - 2, 1, False)
        step(nb - 1, 0, False)
        plsc.subcore_barrier()

        def fflush(j, _):
            r0 = pl.multiple_of(sid * RPT + j * FB, 8)
            pltpu.sync_copy(acc.at[pl.ds(r0, FB)], zfb)
            off = pl.multiple_of((cid * CH + k) * NACC + r0, 8)
            pltpu.sync_copy(zfb, out.at[pl.ds(off, FB)])
            return 0

        lax.fori_loop(0, NFL, fflush, 0)


def _sc_scatter(f_chunks, dst):
    kfn = pl.kernel(
        _sc_scatter_body,
        out_type=jax.ShapeDtypeStruct((2 * CH * NACC, CW), jnp.float32),
        mesh=_sc_mesh(),
        scratch_types=[
            pltpu.VMEM((KE,), jnp.int32),
            pltpu.VMEM((KE,), jnp.int32),
            pltpu.VMEM((KE, CW), jnp.float32),
            pltpu.VMEM((KE, CW), jnp.float32),
            pltpu.VMEM((FB, CW), jnp.float32),
            pltpu.VMEM_SHARED((NACC, CW), jnp.float32),
            pltpu.SemaphoreType.DMA,
            pltpu.SemaphoreType.DMA,
        ],
    )
    return kfn(f_chunks[0], f_chunks[1], f_chunks[2], f_chunks[3], dst)


def _sc_gather_mul_scatter_body(g0, g1, g2, g3, y0, y1, y2, y3, src, dst, out,
                                si0, si1, di0, di1, gb0, gb1, yb0, yb1, acc,
                                semi0, semi1, semy0, semy1, sems0, sems1):
    cid = lax.axis_index("c")
    sid = lax.axis_index("s")
    ebase = cid * (E // 2) + sid * EPT
    gates = [g0, g1, g2, g3]
    ys = [y0, y1, y2, y3]
    sis = [si0, si1]
    dis = [di0, di1]
    gbs = [gb0, gb1]
    ybs = [yb0, yb1]
    semis = [semi0, semi1]
    semys = [semy0, semy1]
    semss = [sems0, sems1]
    nb = EPT // KG
    nfl = RPT // FG
    for k in range(CH):
        def zrow(r, _):
            for g in range(CW // 16):
                yb0[r, pl.ds(g * 16, 16)] = jnp.zeros((16,), jnp.float32)
            return 0

        lax.fori_loop(0, FG, zrow, 0)

        def zflush(j, _):
            r0 = pl.multiple_of(sid * RPT + j * FG, 8)
            pltpu.sync_copy(yb0, acc.at[pl.ds(r0, FG)])
            return 0

        lax.fori_loop(0, nfl, zflush, 0)
        plsc.subcore_barrier()
        gk = gates[k]
        yk = ys[k]

        def issue_idx(b, p):
            e0 = ebase + b * KG
            pltpu.async_copy(src.at[pl.ds(e0, KG)], sis[p], semis[p])
            pltpu.async_copy(dst.at[pl.ds(e0, KG)], dis[p], semis[p])
            pltpu.async_copy(gk.at[pl.ds(e0, KG)], gbs[p], semis[p])

        def drain_idx(b, p):
            e0 = ebase + b * KG
            pltpu.make_async_copy(src.at[pl.ds(e0, KG)], sis[p], semis[p]).wait()
            pltpu.make_async_copy(dst.at[pl.ds(e0, KG)], dis[p], semis[p]).wait()
            pltpu.make_async_copy(gk.at[pl.ds(e0, KG)], gbs[p],
                                  semis[p]).wait()

        def issue_y(p):
            pltpu.async_copy(yk.at[sis[p]], ybs[p], semys[p])

        def drain_y(p):
            pltpu.make_async_copy(yk.at[sis[p]], ybs[p], semys[p]).wait()

        def drain_scatter(p):
            pltpu.make_async_copy(ybs[p], acc.at[dis[p]], semss[p]).wait()

        def mul_scatter(p):
            yb = ybs[p]
            gb = gbs[p]

            def row(r, _):
                for g in range(CW // 16):
                    s = pl.ds(g * 16, 16)
                    yb[r, s] = yb[r, s] * gb[r, s]
                return 0

            lax.fori_loop(0, KG, row, 0)
            pltpu.async_copy(yb, acc.at[dis[p]], semss[p], add=True)

        def step(b, p, pre_idx, pre_y, drain_sc):
            drain_y(p)
            if drain_sc:
                drain_scatter(1 - p)
            if pre_y:
                drain_idx(b + 1, 1 - p)
                issue_y(1 - p)
            mul_scatter(p)
            if pre_idx:
                issue_idx(b + 2, p)

        issue_idx(0, 0)
        drain_idx(0, 0)
        issue_y(0)
        issue_idx(1, 1)
        step(0, 0, True, True, False)

        def pair(t, _):
            b = 2 * t + 1
            step(b, 1, True, True, True)
            step(b + 1, 0, True, True, True)
            return 0

        lax.fori_loop(0, (nb - 3) // 2, pair, 0)
        step(nb - 2, 1, False, True, True)
        step(nb - 1, 0, False, False, True)
        drain_scatter(0)
        plsc.subcore_barrier()

        def fflush(j, _):
            r0 = pl.multiple_of(sid * RPT + j * FG, 8)
            pltpu.sync_copy(acc.at[pl.ds(r0, FG)], yb0)
            off = pl.multiple_of((cid * CH + k) * NACC + r0, 8)
            pltpu.sync_copy(yb0, out.at[pl.ds(off, FG)])
            return 0

        lax.fori_loop(0, nfl, fflush, 0)


def _sc_gather_mul_scatter(g_chunks, y_chunks, src, dst):
    kfn = pl.kernel(
        _sc_gather_mul_scatter_body,
        out_type=jax.ShapeDtypeStruct((2 * CH * NACC, CW), jnp.float32),
        mesh=_sc_mesh(),
        scratch_types=[
            pltpu.VMEM((KG,), jnp.int32),
            pltpu.VMEM((KG,), jnp.int32),
            pltpu.VMEM((KG,), jnp.int32),
            pltpu.VMEM((KG,), jnp.int32),
            pltpu.VMEM((KG, CW), jnp.float32),
            pltpu.VMEM((KG, CW), jnp.float32),
            pltpu.VMEM((KG, CW), jnp.float32),
            pltpu.VMEM((KG, CW), jnp.float32),
            pltpu.VMEM_SHARED((NACC, CW), jnp.float32),
            pltpu.SemaphoreType.DMA,
            pltpu.SemaphoreType.DMA,
            pltpu.SemaphoreType.DMA,
            pltpu.SemaphoreType.DMA,
            pltpu.SemaphoreType.DMA,
            pltpu.SemaphoreType.DMA,
        ],
    )
    return kfn(g_chunks[0], g_chunks[1], g_chunks[2], g_chunks[3],
               y_chunks[0], y_chunks[1], y_chunks[2], y_chunks[3], src, dst)


def _silu(v):
    return v * jax.nn.sigmoid(v)


def _sh16(vec16):
    b = vec16.shape[0]
    r = jnp.sqrt(jnp.sum(vec16 * vec16, axis=1, keepdims=True))
    u = vec16 / jnp.maximum(r, 1e-9)
    x = u[:, 0:1]
    y = u[:, 1:2]
    z = u[:, 2:3]
    sh = jnp.concatenate([
        jnp.ones((b, 1), jnp.float32),
        _S3 * x, _S3 * y, _S3 * z,
        _S15 * x * y, _S15 * y * z,
        (_S5 / 2.0) * (3.0 * z * z - 1.0),
        _S15 * x * z, (_S15 / 2.0) * (x * x - y * y),
        jnp.zeros((b, 7), jnp.float32),
    ], axis=1)
    return sh, r


def _tc_edge_feat_body(ev_ref, wd1_ref, wd2_ref, wsh_ref,
                       f0_ref, f1_ref, f2_ref, f3_ref, rbf_ref):
    ev = ev_ref[...]
    sh, d = _sh16(ev)
    e5 = math.exp(-MAX_R)
    means = e5 + lax.broadcasted_iota(jnp.int32, (1, NBF), 1).astype(
        jnp.float32) * ((1.0 - e5) / (NBF - 1))
    beta = (2.0 / NBF * (1.0 - e5)) ** -2
    fc = 0.5 * (jnp.cos(jnp.pi * jnp.clip(d, 0.0, MAX_R) / MAX_R) + 1.0)
    rbf = fc * jnp.exp(-beta * (jnp.exp(-d) - means) ** 2)
    rbf_ref[...] = rbf
    h = _silu(jnp.dot(rbf, wd1_ref[...], preferred_element_type=jnp.float32))
    dw = jnp.dot(h, wd2_ref[...], preferred_element_type=jnp.float32)
    f = dw * jnp.dot(sh, wsh_ref[...], preferred_element_type=jnp.float32)
    f0_ref[...] = f[:, 0 * CW:1 * CW]
    f1_ref[...] = f[:, 1 * CW:2 * CW]
    f2_ref[...] = f[:, 2 * CW:3 * CW]
    f3_ref[...] = f[:, 3 * CW:4 * CW]


def _tc_edge_feat(evec, wd1, wd2p, wshp):
    be = 512
    fspec = pl.BlockSpec((be, CW), lambda i: (i, 0))
    return pl.pallas_call(
        _tc_edge_feat_body,
        grid=(E // be,),
        in_specs=[
            pl.BlockSpec((be, 16), lambda i: (i, 0)),
            pl.BlockSpec((NBF, 64), lambda i: (0, 0)),
            pl.BlockSpec((64, DP), lambda i: (0, 0)),
            pl.BlockSpec((16, DP), lambda i: (0, 0)),
        ],
        out_specs=[fspec, fspec, fspec, fspec,
                   pl.BlockSpec((be, NBF), lambda i: (i, 0))],
        out_shape=[jax.ShapeDtypeStruct((E, CW), jnp.float32) for _ in range(4)]
        + [jax.ShapeDtypeStruct((E, NBF), jnp.float32)],
    )(evec, wd1, wd2p, wshp)


def _tc_gate_body(rbf_ref, wr1_ref, wr2_ref, g0_ref, g1_ref, g2_ref, g3_ref):
    h = _silu(jnp.dot(rbf_ref[...], wr1_ref[...],
                      preferred_element_type=jnp.float32))
    gfull = _silu(jnp.dot(h, wr2_ref[...], preferred_element_type=jnp.float32))
    g0_ref[...] = gfull[:, 0 * CW:1 * CW]
    g1_ref[...] = gfull[:, 1 * CW:2 * CW]
    g2_ref[...] = gfull[:, 2 * CW:3 * CW]
    g3_ref[...] = gfull[:, 3 * CW:4 * CW]


def _tc_gate(rbf, wr1_i, wr2p_i):
    be = 512
    gspec = pl.BlockSpec((be, CW), lambda i: (i, 0))
    return pl.pallas_call(
        _tc_gate_body,
        grid=(E // be,),
        in_specs=[
            pl.BlockSpec((be, NBF), lambda i: (i, 0)),
            pl.BlockSpec((NBF, 64), lambda i: (0, 0)),
            pl.BlockSpec((64, DP), lambda i: (0, 0)),
        ],
        out_specs=[gspec] * 4,
        out_shape=[jax.ShapeDtypeStruct((E, CW), jnp.float32)
                   for _ in range(4)],
    )(rbf, wr1_i, wr2p_i)


def _tc_x0_body(xemb_ref, part_ref, force_ref, wf_ref, wm_ref,
                x_ref, y0_ref, y1_ref, y2_ref, y3_ref):
    p = part_ref[...]
    agg = jnp.concatenate([p[0, k] + p[1, k] for k in range(CH)], axis=1)
    sh, r = _sh16(force_ref[...])
    fenc = jnp.dot(sh * (r * (1.0 / _S3)), wf_ref[...],
                   preferred_element_type=jnp.float32)
    x = 8.0 * xemb_ref[...] + agg * _INV_SQRT_DEG + fenc
    x_ref[...] = x
    y = jnp.dot(x, wm_ref[...], preferred_element_type=jnp.float32)
    y0_ref[...] = y[:, 0 * CW:1 * CW]
    y1_ref[...] = y[:, 1 * CW:2 * CW]
    y2_ref[...] = y[:, 2 * CW:3 * CW]
    y3_ref[...] = y[:, 3 * CW:4 * CW]


def _tc_x0(xemb, part, forcep, wfp, wm0):
    bn = 400
    yspec = pl.BlockSpec((bn, CW), lambda i: (i, 0))
    return pl.pallas_call(
        _tc_x0_body,
        grid=(N // bn,),
        in_specs=[
            pl.BlockSpec((bn, DP), lambda i: (i, 0)),
            pl.BlockSpec((2, CH, bn, CW), lambda i: (0, 0, i, 0)),
            pl.BlockSpec((bn, 16), lambda i: (i, 0)),
            pl.BlockSpec((16, DP), lambda i: (0, 0)),
            pl.BlockSpec((DP, DP), lambda i: (0, 0)),
        ],
        out_specs=[pl.BlockSpec((bn, DP), lambda i: (i, 0))] + [yspec] * 4,
        out_shape=[jax.ShapeDtypeStruct((N, DP), jnp.float32)]
        + [jax.ShapeDtypeStruct((N, CW), jnp.float32) for _ in range(4)],
    )(xemb, part, forcep, wfp, wm0)


def _tc_update_body(x_ref, part_ref, wu_ref, wm_ref,
                    xn_ref, y0_ref, y1_ref, y2_ref, y3_ref):
    p = part_ref[...]
    agg = jnp.concatenate([p[0, k] + p[1, k] for k in range(CH)], axis=1)
    a = _silu(agg * _INV_SQRT_DEG)
    xn = x_ref[...] + jnp.dot(a, wu_ref[...], preferred_element_type=jnp.float32)
    xn_ref[...] = xn
    y = jnp.dot(xn, wm_ref[...], preferred_element_type=jnp.float32)
    y0_ref[...] = y[:, 0 * CW:1 * CW]
    y1_ref[...] = y[:, 1 * CW:2 * CW]
    y2_ref[...] = y[:, 2 * CW:3 * CW]
    y3_ref[...] = y[:, 3 * CW:4 * CW]


def _tc_update(x, part, wu_i, wm_next):
    bn = 400
    yspec = pl.BlockSpec((bn, CW), lambda i: (i, 0))
    return pl.pallas_call(
        _tc_update_body,
        grid=(N // bn,),
        in_specs=[
            pl.BlockSpec((bn, DP), lambda i: (i, 0)),
            pl.BlockSpec((2, CH, bn, CW), lambda i: (0, 0, i, 0)),
            pl.BlockSpec((DP, DP), lambda i: (0, 0)),
            pl.BlockSpec((DP, DP), lambda i: (0, 0)),
        ],
        out_specs=[pl.BlockSpec((bn, DP), lambda i: (i, 0))] + [yspec] * 4,
        out_shape=[jax.ShapeDtypeStruct((N, DP), jnp.float32)]
        + [jax.ShapeDtypeStruct((N, CW), jnp.float32) for _ in range(4)],
    )(x, part, wu_i, wm_next)


def _tc_update_last_body(x_ref, part_ref, wu_ref, xn_ref):
    p = part_ref[...]
    agg = jnp.concatenate([p[0, k] + p[1, k] for k in range(CH)], axis=1)
    a = _silu(agg * _INV_SQRT_DEG)
    xn_ref[...] = x_ref[...] + jnp.dot(a, wu_ref[...],
                                       preferred_element_type=jnp.float32)


def _tc_update_last(x, part, wu_i):
    bn = 400
    return pl.pallas_call(
        _tc_update_last_body,
        grid=(N // bn,),
        in_specs=[
            pl.BlockSpec((bn, DP), lambda i: (i, 0)),
            pl.BlockSpec((2, CH, bn, CW), lambda i: (0, 0, i, 0)),
            pl.BlockSpec((DP, DP), lambda i: (0, 0)),
        ],
        out_specs=pl.BlockSpec((bn, DP), lambda i: (i, 0)),
        out_shape=jax.ShapeDtypeStruct((N, DP), jnp.float32),
    )(x, part, wu_i)


def _tc_head_body(x_ref, batch_ref, wfeat_ref, we1_ref, we2_ref, wp_ref,
                  energy_ref, pos_ref):
    feat = jnp.dot(x_ref[...], wfeat_ref[...], preferred_element_type=jnp.float32)
    rms = jnp.sqrt(jnp.mean(feat * feat, axis=1, keepdims=True) + 1e-6)
    feat = feat / rms
    scal = feat[:, :512]
    h = _silu(jnp.dot(scal, we1_ref[...], preferred_element_type=jnp.float32))
    node_e = jnp.dot(h, we2_ref[...], preferred_element_type=jnp.float32)
    node_e = node_e * _INV_SQRT_NODES
    seg = lax.broadcasted_iota(jnp.int32, (G, 1), 0)
    onehot = (seg == batch_ref[0]).astype(jnp.float32)
    epart = jnp.dot(onehot, node_e, preferred_element_type=jnp.float32)

    @pl.when(pl.program_id(0) == 0)
    def _():
        energy_ref[...] = epart

    @pl.when(pl.program_id(0) > 0)
    def _():
        energy_ref[...] += epart

    pos_ref[...] = jnp.dot(feat, wp_ref[...], preferred_element_type=jnp.float32)


def _tc_head(x, batchp, wfeatp, we1, we2p, wpp):
    bn = 400
    return pl.pallas_call(
        _tc_head_body,
        grid=(N // bn,),
        in_specs=[
            pl.BlockSpec((bn, DP), lambda i: (i, 0)),
            pl.BlockSpec((1, 1, bn), lambda i: (i, 0, 0)),
            pl.BlockSpec((DP, DF), lambda i: (0, 0)),
            pl.BlockSpec((512, 512), lambda i: (0, 0)),
            pl.BlockSpec((512, CW), lambda i: (0, 0)),
            pl.BlockSpec((DF, CW), lambda i: (0, 0)),
        ],
        out_specs=[
            pl.BlockSpec((G, CW), lambda i: (0, 0)),
            pl.BlockSpec((bn, CW), lambda i: (i, 0)),
        ],
        out_shape=[
            jax.ShapeDtypeStruct((G, CW), jnp.float32),
            jax.ShapeDtypeStruct((N, CW), jnp.float32),
        ],
    )(x, batchp, wfeatp, we1, we2p, wpp)


def kernel(z, pos, batch, edge_index, force, noise_mask, atom_table, W_force,
           W_sh, Wd1, Wd2, Wr1, Wr2, Wmsg, Wupd, Wfeat, We1, We2, Wp):
    f32 = jnp.float32
    src = edge_index[0].astype(jnp.int32)
    dst = edge_index[1].astype(jnp.int32)
    posf = pos.astype(f32)
    posx, posy, posz = posf[:, 0], posf[:, 1], posf[:, 2]
    forcep = jnp.pad(force.astype(f32) * noise_mask[:, None].astype(f32),
                     ((0, 0), (0, 13)))
    zp = jnp.pad(z.astype(jnp.int32), (0, NPAD - N))
    atabp = jnp.pad(atom_table.astype(f32), ((0, 0), (0, DP - D)))
    wshp = jnp.pad(W_sh.astype(f32), ((0, 7), (0, DP - D)))
    wfp = jnp.pad(W_force.astype(f32), ((0, 7), (0, DP - D)))
    wd2p = jnp.pad(Wd2.astype(f32), ((0, 0), (0, DP - D)))
    wr2p = jnp.pad(Wr2.astype(f32), ((0, 0), (0, 0), (0, DP - D)))
    wmsgp = jnp.pad(Wmsg.astype(f32), ((0, 0), (0, DP - D), (0, DP - D)))
    wupdp = jnp.pad(Wupd.astype(f32), ((0, 0), (0, DP - D), (0, DP - D)))
    wfeatp = jnp.pad(Wfeat.astype(f32), ((0, DP - D), (0, 0)))
    we2p = jnp.pad(We2.astype(f32), ((0, 0), (0, CW - 1)))
    wpp = jnp.pad(Wp.astype(f32), ((0, 0), (0, CW - 3)))
    batchp = batch.astype(jnp.int32).reshape(N // 400, 1, 400)

    evx, evy, evz, xemb = _sc_edges_atoms(posx, posy, posz, src, dst, zp, atabp)
    evec = jnp.pad(
        jnp.concatenate([evx[:, None], evy[:, None], evz[:, None]], axis=1),
        ((0, 0), (0, 13)))

    f0, f1, f2, f3, rbf = _tc_edge_feat(evec, Wd1.astype(f32), wd2p, wshp)

    gate_chunks = [_tc_gate(rbf, Wr1[i].astype(f32), wr2p[i]) for i in range(L)]

    deg_part = _sc_scatter((f0, f1, f2, f3), dst).reshape(2, CH, NACC, CW)

    x, y0, y1, y2, y3 = _tc_x0(xemb, deg_part, forcep, wfp, wmsgp[0])
    ys = (y0, y1, y2, y3)

    for i in range(L):
        g_chunks = gate_chunks[i]
        part = _sc_gather_mul_scatter(g_chunks, ys, src, dst)
        part = part.reshape(2, CH, NACC, CW)
        if i < L - 1:
            x, y0, y1, y2, y3 = _tc_update(x, part, wupdp[i], wmsgp[i + 1])
            ys = (y0, y1, y2, y3)
        else:
            x = _tc_update_last(x, part, wupdp[i])

    energy_pad, pos_pad = _tc_head(x, batchp, wfeatp, We1.astype(f32), we2p, wpp)
    return energy_pad[:, :1], pos_pad[:, :3]
